# Initial kernel scaffold; baseline (speedup 1.0000x reference)
import jax
import jax.numpy as jnp
from jax.experimental import pallas as pl


def kernel(x, edge_index, batch, Wq0, bq0, Wk0, bk0, Wv0, bv0, Ws0, bs0, Wq1, bq1, Wk1, bk1, Wv1, bv1, Ws1, bs1, W1, b1, W2, b2):
    raise NotImplementedError("write your pallas kernel here")


# speedup vs baseline: 1.3947x; 1.3947x over previous
import functools

import jax
import jax.numpy as jnp
from jax.experimental import pallas as pl
from jax.experimental.pallas import tpu as pltpu

N_NODES = 10000
HID = 128


def _qkvs_kernel(x_ref, w_ref, b_ref, o_ref):
    o_ref[...] = (
        jnp.dot(x_ref[...], w_ref[...], preferred_element_type=jnp.float32)
        + b_ref[...]
    )


def _qkvs(x, wcat, bcat):
    n = x.shape[0]
    nb = 1000
    return pl.pallas_call(
        _qkvs_kernel,
        grid=(n // nb,),
        in_specs=[
            pl.BlockSpec((nb, x.shape[1]), lambda i: (i, 0)),
            pl.BlockSpec((x.shape[1], wcat.shape[1]), lambda i: (0, 0)),
            pl.BlockSpec((1, wcat.shape[1]), lambda i: (0, 0)),
        ],
        out_specs=pl.BlockSpec((nb, wcat.shape[1]), lambda i: (i, 0)),
        out_shape=jax.ShapeDtypeStruct((n, wcat.shape[1]), jnp.float32),
    )(x, wcat, bcat)


def _pool_mlp_kernel(h_ref, batch_ref, w1_ref, b1_ref, w2_ref, b2_ref,
                     o_ref, acc_ref):
    i = pl.program_id(0)

    @pl.when(i == 0)
    def _():
        acc_ref[...] = jnp.zeros_like(acc_ref)

    b = batch_ref[0, 0]
    g = acc_ref.shape[0]
    onehot = (
        jax.lax.broadcasted_iota(jnp.int32, (g, b.shape[0]), 0)
        == b[None, :]
    ).astype(jnp.float32)
    acc_ref[...] += jnp.dot(onehot, h_ref[...],
                            preferred_element_type=jnp.float32)

    @pl.when(i == pl.num_programs(0) - 1)
    def _():
        gact = jax.nn.relu(
            jnp.dot(acc_ref[...], w1_ref[...],
                    preferred_element_type=jnp.float32) + b1_ref[...]
        )
        o_ref[...] = (
            jnp.dot(gact, w2_ref[...], preferred_element_type=jnp.float32)
            + b2_ref[...]
        )


def _pool_mlp(h, batch, w1, b1, w2, b2, g):
    n, d = h.shape
    nb = 1000
    return pl.pallas_call(
        _pool_mlp_kernel,
        grid=(n // nb,),
        in_specs=[
            pl.BlockSpec((nb, d), lambda i: (i, 0)),
            pl.BlockSpec((1, 1, nb), lambda i: (i, 0, 0)),
            pl.BlockSpec(w1.shape, lambda i: (0, 0)),
            pl.BlockSpec((1, b1.shape[-1]), lambda i: (0, 0)),
            pl.BlockSpec(w2.shape, lambda i: (0, 0)),
            pl.BlockSpec((1, b2.shape[-1]), lambda i: (0, 0)),
        ],
        out_specs=pl.BlockSpec((g, w2.shape[1]), lambda i: (0, 0)),
        out_shape=jax.ShapeDtypeStruct((g, w2.shape[1]), jnp.float32),
        scratch_shapes=[pltpu.VMEM((g, d), jnp.float32)],
    )(h, batch.reshape(n // nb, 1, nb), w1, b1.reshape(1, -1), w2,
      b2.reshape(1, -1))


def _edge_layer(q, k, v, skip, src, dst):
    n = q.shape[0]
    d = q.shape[1]
    logits = jnp.sum(q[dst] * k[src], axis=1) / jnp.sqrt(
        jnp.asarray(d, jnp.float32))
    m = jax.ops.segment_max(logits, dst, num_segments=n)
    m = jnp.where(jnp.isfinite(m), m, 0.0)
    e = jnp.exp(logits - m[dst])
    s = jax.ops.segment_sum(e, dst, num_segments=n)
    agg = jax.ops.segment_sum(v[src] * e[:, None], dst, num_segments=n)
    return jax.nn.relu(agg / (s[:, None] + 1e-16) + skip)


def kernel(x, edge_index, batch,
           Wq0, bq0, Wk0, bk0, Wv0, bv0, Ws0, bs0,
           Wq1, bq1, Wk1, bk1, Wv1, bv1, Ws1, bs1,
           W1, b1, W2, b2):
    src = edge_index[0]
    dst = edge_index[1]
    wcat0 = jnp.concatenate([Wq0, Wk0, Wv0, Ws0], axis=1)
    bcat0 = jnp.concatenate([bq0, bk0, bv0, bs0]).reshape(1, -1)
    wcat1 = jnp.concatenate([Wq1, Wk1, Wv1, Ws1], axis=1)
    bcat1 = jnp.concatenate([bq1, bk1, bv1, bs1]).reshape(1, -1)

    p0 = _qkvs(x, wcat0, bcat0)
    q0, k0, v0, s0 = jnp.split(p0, 4, axis=1)
    h = _edge_layer(q0, k0, v0, s0, src, dst)

    p1 = _qkvs(h, wcat1, bcat1)
    q1, k1, v1, s1 = jnp.split(p1, 4, axis=1)
    h2 = _edge_layer(q1, k1, v1, s1, src, dst)

    return _pool_mlp(h2, batch, W1, b1, W2, b2, 64)


# speedup vs baseline: 8.1014x; 5.8088x over previous
import functools
import math

import jax
import jax.numpy as jnp
from jax import lax
from jax.experimental import pallas as pl
from jax.experimental.pallas import tpu as pltpu
from jax.experimental.pallas import tpu_sc as plsc

N_NODES = 10000
N_EDGES = 320000
D = 128
G = 64
NC, NS = 2, 16
NW = NC * NS
EPT = N_EDGES // NW
CH = 80
NCHUNK = EPT // CH
NG = CH // 16
N_PAD = 10240
ROWS_PT = N_PAD // NS
SCALE = 1.0 / math.sqrt(float(D))

_MESH = plsc.VectorSubcoreMesh(core_axis_name="c", subcore_axis_name="s")
_SC_PARAMS = pltpu.CompilerParams(needs_layout_passes=False)


def _iota16():
    return lax.iota(jnp.int32, 16)


def _take16(x, idx):
    dnums = lax.GatherDimensionNumbers(
        offset_dims=(), collapsed_slice_dims=(0,), start_index_map=(0,))
    return lax.gather(x, idx[:, None], dnums, (1,),
                      mode=lax.GatherScatterMode.PROMISE_IN_BOUNDS)


def _seg_max_scan(keys, vals):
    io = _iota16()
    for sh in (1, 2, 4, 8):
        idx = jnp.maximum(io - sh, 0)
        kv = _take16(keys, idx)
        vv = _take16(vals, idx)
        ok = (io >= sh) & (kv == keys)
        vals = jnp.where(ok, jnp.maximum(vals, vv), vals)
    return vals


def _last_of_run(keys):
    io = _iota16()
    nxt = _take16(keys, jnp.minimum(io + 1, 15))
    return (keys != nxt) | (io == 15)


def _sc_logits_body(q_hbm, k_hbm, src_hbm, dst_hbm, logits_hbm, mpart_hbm,
                    src_v, dst_v, qbuf, kbuf, lbuf, m_local):
    cid = lax.axis_index("c")
    sid = lax.axis_index("s")
    wid = cid * NS + sid
    pltpu.sync_copy(src_hbm.at[wid], src_v)
    pltpu.sync_copy(dst_hbm.at[wid], dst_v)

    neg = jnp.full((16,), -jnp.inf, jnp.float32)

    def init_body(i, _):
        m_local[pl.ds(i * 16, 16)] = neg
        return 0

    lax.fori_loop(0, N_NODES // 16, init_body, 0)

    def chunk_body(c, _):
        pltpu.sync_copy(q_hbm.at[dst_v.at[c]], qbuf)
        pltpu.sync_copy(k_hbm.at[src_v.at[c]], kbuf)

        def group_body(g, _2):
            base = g * 16
            io = _iota16()
            l16 = jnp.zeros((16,), jnp.float32)
            for j in range(16):
                r = base + j
                acc = qbuf[r, pl.ds(0, 16)] * kbuf[r, pl.ds(0, 16)]
                for dv in range(1, D // 16):
                    sl = pl.ds(dv * 16, 16)
                    acc = acc + qbuf[r, sl] * kbuf[r, sl]
                for sh in (1, 2, 4, 8):
                    acc = acc + _take16(acc, io ^ sh)
                l16 = jnp.where(io == j, acc, l16)
            l16 = l16 * SCALE
            lbuf[pl.ds(c * CH + base, 16)] = l16
            d16 = dst_v[c, pl.ds(base, 16)]
            ks, vs = plsc.sort_key_val(d16, l16)
            vs = _seg_max_scan(ks, vs)
            isl = _last_of_run(ks)
            cur = plsc.load_gather(m_local, [ks], mask=isl)
            plsc.store_scatter(m_local, [ks], jnp.maximum(cur, vs), mask=isl)
            return 0

        lax.fori_loop(0, NG, group_body, 0)
        return 0

    lax.fori_loop(0, NCHUNK, chunk_body, 0)
    pltpu.sync_copy(lbuf, logits_hbm.at[pl.ds(wid * EPT, EPT)])
    pltpu.sync_copy(m_local, mpart_hbm.at[wid])


_sc_logits = pl.kernel(
    _sc_logits_body,
    out_type=[
        jax.ShapeDtypeStruct((N_EDGES,), jnp.float32),
        jax.ShapeDtypeStruct((NW, N_NODES), jnp.float32),
    ],
    mesh=_MESH,
    scratch_types=[
        pltpu.VMEM((NCHUNK, CH), jnp.int32),
        pltpu.VMEM((NCHUNK, CH), jnp.int32),
        pltpu.VMEM((CH, D), jnp.float32),
        pltpu.VMEM((CH, D), jnp.float32),
        pltpu.VMEM((EPT,), jnp.float32),
        pltpu.VMEM((N_NODES,), jnp.float32),
    ],
    compiler_params=_SC_PARAMS,
)


def _seg_add_scan(keys, vals):
    io = _iota16()
    for sh in (1, 2, 4, 8):
        idx = jnp.maximum(io - sh, 0)
        kv = _take16(keys, idx)
        vv = _take16(vals, idx)
        ok = (io >= sh) & (kv == keys)
        vals = vals + jnp.where(ok, vv, 0.0)
    return vals


def _sc_agg_body(v_hbm, srcf_hbm, dst_hbm, logits_hbm, m_hbm, zeros_hbm,
                 agg_hbm, spart_hbm,
                 src_c, dst_v, l_c, m_v, vbuf, s_local, agg_sh):
    cid = lax.axis_index("c")
    sid = lax.axis_index("s")
    wid = cid * NS + sid
    pltpu.sync_copy(dst_hbm.at[wid], dst_v)
    pltpu.sync_copy(m_hbm, m_v)

    zero = jnp.zeros((16,), jnp.float32)

    def init_body(i, _):
        s_local[pl.ds(i * 16, 16)] = zero
        return 0

    lax.fori_loop(0, N_NODES // 16, init_body, 0)

    pltpu.sync_copy(zeros_hbm, agg_sh.at[pl.ds(sid * ROWS_PT, ROWS_PT)])
    plsc.subcore_barrier()

    def chunk_body(c, _):
        base_e = wid * EPT + c * CH
        pltpu.sync_copy(srcf_hbm.at[pl.ds(base_e, CH)], src_c)
        pltpu.sync_copy(logits_hbm.at[pl.ds(base_e, CH)], l_c)
        pltpu.sync_copy(v_hbm.at[src_c], vbuf)

        def group_body(g, _2):
            base = g * 16
            l16 = l_c[pl.ds(base, 16)]
            d16 = dst_v[c, pl.ds(base, 16)]
            mg = plsc.load_gather(m_v, [d16])
            e16 = jnp.exp(l16 - mg)
            ks, es = plsc.sort_key_val(d16, e16)
            es = _seg_add_scan(ks, es)
            isl = _last_of_run(ks)
            cur = plsc.load_gather(s_local, [ks], mask=isl)
            plsc.store_scatter(s_local, [ks], cur + es, mask=isl)
            for j in range(16):
                r = base + j
                ebc = _take16(e16, jnp.full((16,), j, jnp.int32))
                for dv in range(D // 16):
                    sl = pl.ds(dv * 16, 16)
                    vbuf[r, sl] = vbuf[r, sl] * ebc
            return 0

        lax.fori_loop(0, NG, group_body, 0)
        pltpu.sync_copy(vbuf, agg_sh.at[dst_v.at[c]], add=True)
        return 0

    lax.fori_loop(0, NCHUNK, chunk_body, 0)
    pltpu.sync_copy(s_local, spart_hbm.at[wid])
    plsc.subcore_barrier()
    sl = pl.ds(sid * ROWS_PT, ROWS_PT)
    pltpu.sync_copy(agg_sh.at[sl], agg_hbm.at[cid, sl])


_sc_agg = pl.kernel(
    _sc_agg_body,
    out_type=[
        jax.ShapeDtypeStruct((NC, N_PAD, D), jnp.float32),
        jax.ShapeDtypeStruct((NW, N_NODES), jnp.float32),
    ],
    mesh=_MESH,
    scratch_types=[
        pltpu.VMEM((CH,), jnp.int32),
        pltpu.VMEM((NCHUNK, CH), jnp.int32),
        pltpu.VMEM((CH,), jnp.float32),
        pltpu.VMEM((N_NODES,), jnp.float32),
        pltpu.VMEM((CH, D), jnp.float32),
        pltpu.VMEM((N_NODES,), jnp.float32),
        pltpu.VMEM_SHARED((N_PAD, D), jnp.float32),
    ],
    compiler_params=_SC_PARAMS,
)


def _proj_kernel(x_ref, wq_ref, bq_ref, wk_ref, bk_ref, wv_ref, bv_ref,
                 ws_ref, bs_ref, q_ref, k_ref, vx_ref, s_ref):
    x = x_ref[...]
    q_ref[...] = jnp.dot(x, wq_ref[...],
                         preferred_element_type=jnp.float32) + bq_ref[...]
    k_ref[...] = jnp.dot(x, wk_ref[...],
                         preferred_element_type=jnp.float32) + bk_ref[...]
    vx_ref[...] = jnp.dot(x, wv_ref[...],
                          preferred_element_type=jnp.float32) + bv_ref[...]
    s_ref[...] = jnp.dot(x, ws_ref[...],
                         preferred_element_type=jnp.float32) + bs_ref[...]


def _proj(x, wq, bq, wk, bk, wv, bv, ws, bs):
    n = x.shape[0]
    nb = 2000
    wspec = pl.BlockSpec((D, D), lambda i: (0, 0))
    bspec = pl.BlockSpec((1, D), lambda i: (0, 0))
    return pl.pallas_call(
        _proj_kernel,
        grid=(n // nb,),
        in_specs=[pl.BlockSpec((nb, D), lambda i: (i, 0)),
                  wspec, bspec, wspec, bspec, wspec, bspec, wspec, bspec],
        out_specs=[pl.BlockSpec((nb, D), lambda i: (i, 0)),
                   pl.BlockSpec((nb, D), lambda i: (i, 0)),
                   pl.BlockSpec((nb, D), lambda i: (i, 0)),
                   pl.BlockSpec((nb, D), lambda i: (i, 0))],
        out_shape=[jax.ShapeDtypeStruct((n, D), jnp.float32),
                   jax.ShapeDtypeStruct((n, D), jnp.float32),
                   jax.ShapeDtypeStruct((n, D), jnp.float32),
                   jax.ShapeDtypeStruct((n, D), jnp.float32)],
    )(x, wq, bq.reshape(1, D), wk, bk.reshape(1, D), wv, bv.reshape(1, D),
      ws, bs.reshape(1, D))


def _maxreduce_kernel(mp_ref, m_ref):
    m = jnp.max(mp_ref[...], axis=0, keepdims=True)
    m_ref[...] = jnp.where(jnp.isfinite(m), m, 0.0)


def _maxreduce(mpart):
    return pl.pallas_call(
        _maxreduce_kernel,
        out_shape=jax.ShapeDtypeStruct((1, N_NODES), jnp.float32),
    )(mpart)


def _sumreduce_kernel(sp_ref, s_ref):
    s_ref[...] = jnp.sum(sp_ref[...], axis=0)[:, None]


def _sumreduce(spart):
    return pl.pallas_call(
        _sumreduce_kernel,
        out_shape=jax.ShapeDtypeStruct((N_NODES, 1), jnp.float32),
    )(spart)


def _epilogue_h(agg, scol, skip):
    a = agg[0] + agg[1]
    return jax.nn.relu(a / (scol + 1e-16) + skip)


def _epi_proj_kernel(agg_ref, sp_ref, skip_ref, wq_ref, bq_ref, wk_ref,
                     bk_ref, wv_ref, bv_ref, ws_ref, bs_ref,
                     q_ref, k_ref, vx_ref, s_ref):
    h = _epilogue_h(agg_ref[...], sp_ref[...], skip_ref[...])
    q_ref[...] = jnp.dot(h, wq_ref[...],
                         preferred_element_type=jnp.float32) + bq_ref[...]
    k_ref[...] = jnp.dot(h, wk_ref[...],
                         preferred_element_type=jnp.float32) + bk_ref[...]
    vx_ref[...] = jnp.dot(h, wv_ref[...],
                          preferred_element_type=jnp.float32) + bv_ref[...]
    s_ref[...] = jnp.dot(h, ws_ref[...],
                         preferred_element_type=jnp.float32) + bs_ref[...]


def _epi_proj(agg, spart, skip, wq, bq, wk, bk, wv, bv, ws, bs):
    n = skip.shape[0]
    nb = 2000
    wspec = pl.BlockSpec((D, D), lambda i: (0, 0))
    bspec = pl.BlockSpec((1, D), lambda i: (0, 0))
    return pl.pallas_call(
        _epi_proj_kernel,
        grid=(n // nb,),
        in_specs=[pl.BlockSpec((NC, nb, D), lambda i: (0, i, 0)),
                  pl.BlockSpec((nb, 1), lambda i: (i, 0)),
                  pl.BlockSpec((nb, D), lambda i: (i, 0)),
                  wspec, bspec, wspec, bspec, wspec, bspec, wspec, bspec],
        out_specs=[pl.BlockSpec((nb, D), lambda i: (i, 0)),
                   pl.BlockSpec((nb, D), lambda i: (i, 0)),
                   pl.BlockSpec((nb, D), lambda i: (i, 0)),
                   pl.BlockSpec((nb, D), lambda i: (i, 0))],
        out_shape=[jax.ShapeDtypeStruct((n, D), jnp.float32),
                   jax.ShapeDtypeStruct((n, D), jnp.float32),
                   jax.ShapeDtypeStruct((n, D), jnp.float32),
                   jax.ShapeDtypeStruct((n, D), jnp.float32)],
    )(agg, spart, skip, wq, bq.reshape(1, D), wk, bk.reshape(1, D),
      wv, bv.reshape(1, D), ws, bs.reshape(1, D))


def _pool_mlp_kernel(agg_ref, sp_ref, skip_ref, batch_ref, w1_ref, b1_ref,
                     w2_ref, b2_ref, o_ref, acc_ref):
    i = pl.program_id(0)

    @pl.when(i == 0)
    def _():
        acc_ref[...] = jnp.zeros_like(acc_ref)

    h2 = _epilogue_h(agg_ref[...], sp_ref[...], skip_ref[...])
    b = batch_ref[0, 0]
    onehot = (jax.lax.broadcasted_iota(jnp.int32, (G, b.shape[0]), 0)
              == b[None, :]).astype(jnp.float32)
    acc_ref[...] += jnp.dot(onehot, h2, preferred_element_type=jnp.float32)

    @pl.when(i == pl.num_programs(0) - 1)
    def _():
        gact = jax.nn.relu(
            jnp.dot(acc_ref[...], w1_ref[...],
                    preferred_element_type=jnp.float32) + b1_ref[...])
        o_ref[...] = jnp.dot(gact, w2_ref[...],
                             preferred_element_type=jnp.float32) + b2_ref[...]


def _pool_mlp(agg, spart, skip, batch, w1, b1, w2, b2):
    n = skip.shape[0]
    nb = 2000
    return pl.pallas_call(
        _pool_mlp_kernel,
        grid=(n // nb,),
        in_specs=[
            pl.BlockSpec((NC, nb, D), lambda i: (0, i, 0)),
            pl.BlockSpec((nb, 1), lambda i: (i, 0)),
            pl.BlockSpec((nb, D), lambda i: (i, 0)),
            pl.BlockSpec((1, 1, nb), lambda i: (i, 0, 0)),
            pl.BlockSpec(w1.shape, lambda i: (0, 0)),
            pl.BlockSpec((1, D), lambda i: (0, 0)),
            pl.BlockSpec(w2.shape, lambda i: (0, 0)),
            pl.BlockSpec((1, D), lambda i: (0, 0)),
        ],
        out_specs=pl.BlockSpec((G, D), lambda i: (0, 0)),
        out_shape=jax.ShapeDtypeStruct((G, D), jnp.float32),
        scratch_shapes=[pltpu.VMEM((G, D), jnp.float32)],
    )(agg, spart, skip, batch.reshape(n // nb, 1, nb), w1, b1.reshape(1, D),
      w2, b2.reshape(1, D))


def _edge_layer_sc(q, k, v, src_f, src_t, dst_t, zeros_rows):
    logits, mpart = _sc_logits(q, k, src_t, dst_t)
    m = _maxreduce(mpart).reshape(N_NODES)
    agg, spart = _sc_agg(v, src_f, dst_t, logits, m, zeros_rows)
    return agg, _sumreduce(spart)


def kernel(x, edge_index, batch,
           Wq0, bq0, Wk0, bk0, Wv0, bv0, Ws0, bs0,
           Wq1, bq1, Wk1, bk1, Wv1, bv1, Ws1, bs1,
           W1, b1, W2, b2):
    src_f = edge_index[0]
    src_t = edge_index[0].reshape(NW, NCHUNK, CH)
    dst_t = edge_index[1].reshape(NW, NCHUNK, CH)
    zeros_rows = jnp.zeros((ROWS_PT, D), jnp.float32)

    q0, k0, vx0, skip0 = _proj(x, Wq0, bq0, Wk0, bk0, Wv0, bv0, Ws0, bs0)
    agg0, sp0 = _edge_layer_sc(q0, k0, vx0, src_f, src_t, dst_t, zeros_rows)

    q1, k1, vx1, skip1 = _epi_proj(agg0, sp0, skip0, Wq1, bq1, Wk1, bk1,
                                   Wv1, bv1, Ws1, bs1)
    agg1, sp1 = _edge_layer_sc(q1, k1, vx1, src_f, src_t, dst_t, zeros_rows)

    return _pool_mlp(agg1, sp1, skip1, batch, W1, b1, W2, b2)


# speedup vs baseline: 12.7203x; 1.5701x over previous
import functools
import math

import jax
import jax.numpy as jnp
from jax import lax
from jax.experimental import pallas as pl
from jax.experimental.pallas import tpu as pltpu
from jax.experimental.pallas import tpu_sc as plsc

N_NODES = 10000
N_EDGES = 320000
D = 128
G = 64
NC, NS = 2, 16
NW = NC * NS
EPT = N_EDGES // NW
CH = 80
NCHUNK = EPT // CH
NG = CH // 16
N_PAD = 10240
ROWS_PT = N_PAD // NS
SCALE = 1.0 / math.sqrt(float(D))

_MESH = plsc.VectorSubcoreMesh(core_axis_name="c", subcore_axis_name="s")
_SC_PARAMS = pltpu.CompilerParams(needs_layout_passes=False)


def _iota16():
    return lax.iota(jnp.int32, 16)


def _take16(x, idx):
    dnums = lax.GatherDimensionNumbers(
        offset_dims=(), collapsed_slice_dims=(0,), start_index_map=(0,))
    return lax.gather(x, idx[:, None], dnums, (1,),
                      mode=lax.GatherScatterMode.PROMISE_IN_BOUNDS)


def _seg_max_scan(keys, vals):
    io = _iota16()
    for sh in (1, 2, 4, 8):
        idx = jnp.maximum(io - sh, 0)
        kv = _take16(keys, idx)
        vv = _take16(vals, idx)
        ok = (io >= sh) & (kv == keys)
        vals = jnp.where(ok, jnp.maximum(vals, vv), vals)
    return vals


def _last_of_run(keys):
    io = _iota16()
    nxt = _take16(keys, jnp.minimum(io + 1, 15))
    return (keys != nxt) | (io == 15)


def _sc_logits_body(q_hbm, k_hbm, src_hbm, dst_hbm, logits_hbm, mpart_hbm,
                    src_v, dst_v, qbuf0, qbuf1, kbuf0, kbuf1, lbuf, m_local,
                    semq0, semq1, semk0, semk1):
    cid = lax.axis_index("c")
    sid = lax.axis_index("s")
    wid = cid * NS + sid
    qbufs = (qbuf0, qbuf1)
    kbufs = (kbuf0, kbuf1)
    semqs = (semq0, semq1)
    semks = (semk0, semk1)
    pltpu.sync_copy(src_hbm.at[wid], src_v)
    pltpu.sync_copy(dst_hbm.at[wid], dst_v)

    neg = jnp.full((16,), -jnp.inf, jnp.float32)

    def init_body(i, _):
        m_local[pl.ds(i * 16, 16)] = neg
        return 0

    lax.fori_loop(0, N_NODES // 16, init_body, 0)

    def start_gathers(cc, b):
        pltpu.async_copy(q_hbm.at[dst_v.at[cc]], qbufs[b], semqs[b])
        pltpu.async_copy(k_hbm.at[src_v.at[cc]], kbufs[b], semks[b])

    def wait_gathers(cc, b):
        pltpu.make_async_copy(q_hbm.at[dst_v.at[cc]], qbufs[b],
                              semqs[b]).wait()
        pltpu.make_async_copy(k_hbm.at[src_v.at[cc]], kbufs[b],
                              semks[b]).wait()

    for b in range(2):
        start_gathers(b, b)

    def compute_chunk(cc, b):
        qbuf = qbufs[b]
        kbuf = kbufs[b]

        def group_body(g, _2):
            base = g * 16
            io = _iota16()
            l16 = jnp.zeros((16,), jnp.float32)
            for j in range(16):
                r = base + j
                acc = qbuf[r, pl.ds(0, 16)] * kbuf[r, pl.ds(0, 16)]
                for dv in range(1, D // 16):
                    sl = pl.ds(dv * 16, 16)
                    acc = acc + qbuf[r, sl] * kbuf[r, sl]
                for sh in (1, 2, 4, 8):
                    acc = acc + _take16(acc, io ^ sh)
                l16 = jnp.where(io == j, acc, l16)
            l16 = l16 * SCALE
            lbuf[pl.ds(cc * CH + base, 16)] = l16
            d16 = dst_v[cc, pl.ds(base, 16)]
            ks, vs = plsc.sort_key_val(d16, l16)
            vs = _seg_max_scan(ks, vs)
            isl = _last_of_run(ks)
            cur = plsc.load_gather(m_local, [ks], mask=isl)
            plsc.store_scatter(m_local, [ks], jnp.maximum(cur, vs), mask=isl)
            return 0

        lax.fori_loop(0, NG, group_body, 0)

    @pl.loop(0, NCHUNK, step=2)
    def _chunks(c):
        for b in range(2):
            cc = c + b
            live = (cc < NCHUNK) if b else True

            def do_chunk():
                wait_gathers(cc, b)
                compute_chunk(cc, b)

                @pl.when(cc + 2 < NCHUNK)
                def _():
                    start_gathers(cc + 2, b)

            if b:
                pl.when(live)(do_chunk)
            else:
                do_chunk()

    pltpu.sync_copy(lbuf, logits_hbm.at[pl.ds(wid * EPT, EPT)])
    pltpu.sync_copy(m_local, mpart_hbm.at[wid])


_sc_logits = pl.kernel(
    _sc_logits_body,
    out_type=[
        jax.ShapeDtypeStruct((N_EDGES,), jnp.float32),
        jax.ShapeDtypeStruct((NW, N_NODES), jnp.float32),
    ],
    mesh=_MESH,
    scratch_types=[
        pltpu.VMEM((NCHUNK, CH), jnp.int32),
        pltpu.VMEM((NCHUNK, CH), jnp.int32),
        pltpu.VMEM((CH, D), jnp.float32),
        pltpu.VMEM((CH, D), jnp.float32),
        pltpu.VMEM((CH, D), jnp.float32),
        pltpu.VMEM((CH, D), jnp.float32),
        pltpu.VMEM((EPT,), jnp.float32),
        pltpu.VMEM((N_NODES,), jnp.float32),
        pltpu.SemaphoreType.DMA,
        pltpu.SemaphoreType.DMA,
        pltpu.SemaphoreType.DMA,
        pltpu.SemaphoreType.DMA,
    ],
    compiler_params=_SC_PARAMS,
)


def _seg_add_scan(keys, vals):
    io = _iota16()
    for sh in (1, 2, 4, 8):
        idx = jnp.maximum(io - sh, 0)
        kv = _take16(keys, idx)
        vv = _take16(vals, idx)
        ok = (io >= sh) & (kv == keys)
        vals = vals + jnp.where(ok, vv, 0.0)
    return vals


def _sc_agg_body(v_hbm, srcf_hbm, dstf_hbm, dst3_hbm, logits_hbm, m_hbm,
                 zeros_hbm, agg_hbm, spart_hbm,
                 src_c0, src_c1, dst_c0, dst_c1, l_c0, l_c1,
                 vbuf0, vbuf1, m_v, s_local, agg_sh,
                 semg0, semg1, semsc0, semsc1):
    cid = lax.axis_index("c")
    sid = lax.axis_index("s")
    wid = cid * NS + sid
    src_cs = (src_c0, src_c1)
    dst_cs = (dst_c0, dst_c1)
    l_cs = (l_c0, l_c1)
    vbufs = (vbuf0, vbuf1)
    semgs = (semg0, semg1)
    semscs = (semsc0, semsc1)
    pltpu.sync_copy(m_hbm, m_v)

    zero = jnp.zeros((16,), jnp.float32)

    def init_body(i, _):
        s_local[pl.ds(i * 16, 16)] = zero
        return 0

    lax.fori_loop(0, N_NODES // 16, init_body, 0)

    pltpu.sync_copy(zeros_hbm, agg_sh.at[pl.ds(sid * ROWS_PT, ROWS_PT)])
    plsc.subcore_barrier()

    def load_smalls_and_gather(cc, b):
        base_e = wid * EPT + cc * CH
        pltpu.sync_copy(srcf_hbm.at[pl.ds(base_e, CH)], src_cs[b])
        pltpu.sync_copy(logits_hbm.at[pl.ds(base_e, CH)], l_cs[b])
        pltpu.sync_copy(dst3_hbm.at[wid * NCHUNK + cc], dst_cs[b])
        pltpu.async_copy(v_hbm.at[src_cs[b]], vbufs[b], semgs[b])

    for b in range(2):
        load_smalls_and_gather(b, b)

    def compute_chunk(b):
        vbuf = vbufs[b]
        l_c = l_cs[b]
        dst_c = dst_cs[b]

        def group_body(g, _2):
            base = g * 16
            l16 = l_c[pl.ds(base, 16)]
            d16 = dst_c[0, pl.ds(base, 16)]
            mg = plsc.load_gather(m_v, [d16])
            e16 = jnp.exp(l16 - mg)
            ks, es = plsc.sort_key_val(d16, e16)
            es = _seg_add_scan(ks, es)
            isl = _last_of_run(ks)
            cur = plsc.load_gather(s_local, [ks], mask=isl)
            plsc.store_scatter(s_local, [ks], cur + es, mask=isl)
            for j in range(16):
                r = base + j
                ebc = _take16(e16, jnp.full((16,), j, jnp.int32))
                for dv in range(D // 16):
                    sl = pl.ds(dv * 16, 16)
                    vbuf[r, sl] = vbuf[r, sl] * ebc
            return 0

        lax.fori_loop(0, NG, group_body, 0)

    @pl.loop(0, NCHUNK, step=2)
    def _chunks(c):
        for b in range(2):
            cc = c + b
            live = (cc < NCHUNK) if b else True

            def do_chunk():
                pltpu.make_async_copy(v_hbm.at[src_cs[b]], vbufs[b],
                                      semgs[b]).wait()
                compute_chunk(b)
                pltpu.async_copy(vbufs[b], agg_sh.at[dst_cs[b].at[0]],
                                 semscs[b], add=True)

                @pl.when(cc + 2 < NCHUNK)
                def _():
                    pltpu.make_async_copy(vbufs[b],
                                          agg_sh.at[dst_cs[b].at[0]],
                                          semscs[b]).wait()
                    load_smalls_and_gather(cc + 2, b)

            if b:
                pl.when(live)(do_chunk)
            else:
                do_chunk()

    for b in range(2):
        pltpu.make_async_copy(vbufs[b], agg_sh.at[dst_cs[b].at[0]],
                              semscs[b]).wait()

    pltpu.sync_copy(s_local, spart_hbm.at[wid])
    plsc.subcore_barrier()
    sl = pl.ds(sid * ROWS_PT, ROWS_PT)
    pltpu.sync_copy(agg_sh.at[sl], agg_hbm.at[cid, sl])


_sc_agg = pl.kernel(
    _sc_agg_body,
    out_type=[
        jax.ShapeDtypeStruct((NC, N_PAD, D), jnp.float32),
        jax.ShapeDtypeStruct((NW, N_NODES), jnp.float32),
    ],
    mesh=_MESH,
    scratch_types=[
        pltpu.VMEM((CH,), jnp.int32),
        pltpu.VMEM((CH,), jnp.int32),
        pltpu.VMEM((1, CH), jnp.int32),
        pltpu.VMEM((1, CH), jnp.int32),
        pltpu.VMEM((CH,), jnp.float32),
        pltpu.VMEM((CH,), jnp.float32),
        pltpu.VMEM((CH, D), jnp.float32),
        pltpu.VMEM((CH, D), jnp.float32),
        pltpu.VMEM((N_NODES,), jnp.float32),
        pltpu.VMEM((N_NODES,), jnp.float32),
        pltpu.VMEM_SHARED((N_PAD, D), jnp.float32),
        pltpu.SemaphoreType.DMA,
        pltpu.SemaphoreType.DMA,
        pltpu.SemaphoreType.DMA,
        pltpu.SemaphoreType.DMA,
    ],
    compiler_params=_SC_PARAMS,
)


def _proj_kernel(x_ref, wq_ref, bq_ref, wk_ref, bk_ref, wv_ref, bv_ref,
                 ws_ref, bs_ref, q_ref, k_ref, vx_ref, s_ref):
    x = x_ref[...]
    q_ref[...] = jnp.dot(x, wq_ref[...],
                         preferred_element_type=jnp.float32) + bq_ref[...]
    k_ref[...] = jnp.dot(x, wk_ref[...],
                         preferred_element_type=jnp.float32) + bk_ref[...]
    vx_ref[...] = jnp.dot(x, wv_ref[...],
                          preferred_element_type=jnp.float32) + bv_ref[...]
    s_ref[...] = jnp.dot(x, ws_ref[...],
                         preferred_element_type=jnp.float32) + bs_ref[...]


def _proj(x, wq, bq, wk, bk, wv, bv, ws, bs):
    n = x.shape[0]
    nb = 2000
    wspec = pl.BlockSpec((D, D), lambda i: (0, 0))
    bspec = pl.BlockSpec((1, D), lambda i: (0, 0))
    return pl.pallas_call(
        _proj_kernel,
        grid=(n // nb,),
        in_specs=[pl.BlockSpec((nb, D), lambda i: (i, 0)),
                  wspec, bspec, wspec, bspec, wspec, bspec, wspec, bspec],
        out_specs=[pl.BlockSpec((nb, D), lambda i: (i, 0)),
                   pl.BlockSpec((nb, D), lambda i: (i, 0)),
                   pl.BlockSpec((nb, D), lambda i: (i, 0)),
                   pl.BlockSpec((nb, D), lambda i: (i, 0))],
        out_shape=[jax.ShapeDtypeStruct((n, D), jnp.float32),
                   jax.ShapeDtypeStruct((n, D), jnp.float32),
                   jax.ShapeDtypeStruct((n, D), jnp.float32),
                   jax.ShapeDtypeStruct((n, D), jnp.float32)],
    )(x, wq, bq.reshape(1, D), wk, bk.reshape(1, D), wv, bv.reshape(1, D),
      ws, bs.reshape(1, D))


def _maxreduce_kernel(mp_ref, m_ref):
    m = jnp.max(mp_ref[...], axis=0, keepdims=True)
    m_ref[...] = jnp.where(jnp.isfinite(m), m, 0.0)


def _maxreduce(mpart):
    return pl.pallas_call(
        _maxreduce_kernel,
        out_shape=jax.ShapeDtypeStruct((1, N_NODES), jnp.float32),
    )(mpart)


def _sumreduce_kernel(sp_ref, s_ref):
    s_ref[...] = jnp.sum(sp_ref[...], axis=0)[:, None]


def _sumreduce(spart):
    return pl.pallas_call(
        _sumreduce_kernel,
        out_shape=jax.ShapeDtypeStruct((N_NODES, 1), jnp.float32),
    )(spart)


def _epilogue_h(agg, scol, skip):
    a = agg[0] + agg[1]
    return jax.nn.relu(a / (scol + 1e-16) + skip)


def _epi_proj_kernel(agg_ref, sp_ref, skip_ref, wq_ref, bq_ref, wk_ref,
                     bk_ref, wv_ref, bv_ref, ws_ref, bs_ref,
                     q_ref, k_ref, vx_ref, s_ref):
    h = _epilogue_h(agg_ref[...], sp_ref[...], skip_ref[...])
    q_ref[...] = jnp.dot(h, wq_ref[...],
                         preferred_element_type=jnp.float32) + bq_ref[...]
    k_ref[...] = jnp.dot(h, wk_ref[...],
                         preferred_element_type=jnp.float32) + bk_ref[...]
    vx_ref[...] = jnp.dot(h, wv_ref[...],
                          preferred_element_type=jnp.float32) + bv_ref[...]
    s_ref[...] = jnp.dot(h, ws_ref[...],
                         preferred_element_type=jnp.float32) + bs_ref[...]


def _epi_proj(agg, spart, skip, wq, bq, wk, bk, wv, bv, ws, bs):
    n = skip.shape[0]
    nb = 2000
    wspec = pl.BlockSpec((D, D), lambda i: (0, 0))
    bspec = pl.BlockSpec((1, D), lambda i: (0, 0))
    return pl.pallas_call(
        _epi_proj_kernel,
        grid=(n // nb,),
        in_specs=[pl.BlockSpec((NC, nb, D), lambda i: (0, i, 0)),
                  pl.BlockSpec((nb, 1), lambda i: (i, 0)),
                  pl.BlockSpec((nb, D), lambda i: (i, 0)),
                  wspec, bspec, wspec, bspec, wspec, bspec, wspec, bspec],
        out_specs=[pl.BlockSpec((nb, D), lambda i: (i, 0)),
                   pl.BlockSpec((nb, D), lambda i: (i, 0)),
                   pl.BlockSpec((nb, D), lambda i: (i, 0)),
                   pl.BlockSpec((nb, D), lambda i: (i, 0))],
        out_shape=[jax.ShapeDtypeStruct((n, D), jnp.float32),
                   jax.ShapeDtypeStruct((n, D), jnp.float32),
                   jax.ShapeDtypeStruct((n, D), jnp.float32),
                   jax.ShapeDtypeStruct((n, D), jnp.float32)],
    )(agg, spart, skip, wq, bq.reshape(1, D), wk, bk.reshape(1, D),
      wv, bv.reshape(1, D), ws, bs.reshape(1, D))


def _pool_mlp_kernel(agg_ref, sp_ref, skip_ref, batch_ref, w1_ref, b1_ref,
                     w2_ref, b2_ref, o_ref, acc_ref):
    i = pl.program_id(0)

    @pl.when(i == 0)
    def _():
        acc_ref[...] = jnp.zeros_like(acc_ref)

    h2 = _epilogue_h(agg_ref[...], sp_ref[...], skip_ref[...])
    b = batch_ref[0, 0]
    onehot = (jax.lax.broadcasted_iota(jnp.int32, (G, b.shape[0]), 0)
              == b[None, :]).astype(jnp.float32)
    acc_ref[...] += jnp.dot(onehot, h2, preferred_element_type=jnp.float32)

    @pl.when(i == pl.num_programs(0) - 1)
    def _():
        gact = jax.nn.relu(
            jnp.dot(acc_ref[...], w1_ref[...],
                    preferred_element_type=jnp.float32) + b1_ref[...])
        o_ref[...] = jnp.dot(gact, w2_ref[...],
                             preferred_element_type=jnp.float32) + b2_ref[...]


def _pool_mlp(agg, spart, skip, batch, w1, b1, w2, b2):
    n = skip.shape[0]
    nb = 2000
    return pl.pallas_call(
        _pool_mlp_kernel,
        grid=(n // nb,),
        in_specs=[
            pl.BlockSpec((NC, nb, D), lambda i: (0, i, 0)),
            pl.BlockSpec((nb, 1), lambda i: (i, 0)),
            pl.BlockSpec((nb, D), lambda i: (i, 0)),
            pl.BlockSpec((1, 1, nb), lambda i: (i, 0, 0)),
            pl.BlockSpec(w1.shape, lambda i: (0, 0)),
            pl.BlockSpec((1, D), lambda i: (0, 0)),
            pl.BlockSpec(w2.shape, lambda i: (0, 0)),
            pl.BlockSpec((1, D), lambda i: (0, 0)),
        ],
        out_specs=pl.BlockSpec((G, D), lambda i: (0, 0)),
        out_shape=jax.ShapeDtypeStruct((G, D), jnp.float32),
        scratch_shapes=[pltpu.VMEM((G, D), jnp.float32)],
    )(agg, spart, skip, batch.reshape(n // nb, 1, nb), w1, b1.reshape(1, D),
      w2, b2.reshape(1, D))


def _edge_layer_sc(q, k, v, src_f, dst_f, src_t, dst_t, dst3, zeros_rows):
    logits, mpart = _sc_logits(q, k, src_t, dst_t)
    m = _maxreduce(mpart).reshape(N_NODES)
    agg, spart = _sc_agg(v, src_f, dst_f, dst3, logits, m, zeros_rows)
    return agg, _sumreduce(spart)


def kernel(x, edge_index, batch,
           Wq0, bq0, Wk0, bk0, Wv0, bv0, Ws0, bs0,
           Wq1, bq1, Wk1, bk1, Wv1, bv1, Ws1, bs1,
           W1, b1, W2, b2):
    src_f = edge_index[0]
    dst_f = edge_index[1]
    src_t = edge_index[0].reshape(NW, NCHUNK, CH)
    dst_t = edge_index[1].reshape(NW, NCHUNK, CH)
    dst3 = edge_index[1].reshape(NW * NCHUNK, 1, CH)
    zeros_rows = jnp.zeros((ROWS_PT, D), jnp.float32)

    q0, k0, vx0, skip0 = _proj(x, Wq0, bq0, Wk0, bk0, Wv0, bv0, Ws0, bs0)
    agg0, sp0 = _edge_layer_sc(q0, k0, vx0, src_f, dst_f, src_t, dst_t,
                               dst3, zeros_rows)

    q1, k1, vx1, skip1 = _epi_proj(agg0, sp0, skip0, Wq1, bq1, Wk1, bk1,
                                   Wv1, bv1, Ws1, bs1)
    agg1, sp1 = _edge_layer_sc(q1, k1, vx1, src_f, dst_f, src_t, dst_t,
                               dst3, zeros_rows)

    return _pool_mlp(agg1, sp1, skip1, batch, W1, b1, W2, b2)


# speedup vs baseline: 17.1307x; 1.3467x over previous
import functools
import math

import jax
import jax.numpy as jnp
from jax import lax
from jax.experimental import pallas as pl
from jax.experimental.pallas import tpu as pltpu
from jax.experimental.pallas import tpu_sc as plsc

N_NODES = 10000
N_EDGES = 320000
D = 128
G = 64
NC, NS = 2, 16
NW = NC * NS
EPT = N_EDGES // NW
CH = 80
NCHUNK = EPT // CH
NG = CH // 16
N_PAD = 10240
ROWS_PT = N_PAD // NS
SCALE = 1.0 / math.sqrt(float(D))

_MESH = plsc.VectorSubcoreMesh(core_axis_name="c", subcore_axis_name="s")
_SC_PARAMS = pltpu.CompilerParams(needs_layout_passes=False)


def _iota16():
    return lax.iota(jnp.int32, 16)


def _take16(x, idx):
    dnums = lax.GatherDimensionNumbers(
        offset_dims=(), collapsed_slice_dims=(0,), start_index_map=(0,))
    return lax.gather(x, idx[:, None], dnums, (1,),
                      mode=lax.GatherScatterMode.PROMISE_IN_BOUNDS)


def _seg_max_scan(keys, vals):
    io = _iota16()
    for sh in (1, 2, 4, 8):
        idx = jnp.maximum(io - sh, 0)
        kv = _take16(keys, idx)
        vv = _take16(vals, idx)
        ok = (io >= sh) & (kv == keys)
        vals = jnp.where(ok, jnp.maximum(vals, vv), vals)
    return vals


def _last_of_run(keys):
    io = _iota16()
    nxt = _take16(keys, jnp.minimum(io + 1, 15))
    return (keys != nxt) | (io == 15)


def _sc_logits_body(q_hbm, k_hbm, src_hbm, dst_hbm, logits_hbm, mpart_hbm,
                    src_v, dst_v, qbuf0, qbuf1, qbuf2, kbuf0, kbuf1, kbuf2,
                    lbuf, m_local,
                    semq0, semq1, semq2, semk0, semk1, semk2):
    cid = lax.axis_index("c")
    sid = lax.axis_index("s")
    wid = cid * NS + sid
    qbufs = (qbuf0, qbuf1, qbuf2)
    kbufs = (kbuf0, kbuf1, kbuf2)
    semqs = (semq0, semq1, semq2)
    semks = (semk0, semk1, semk2)
    pltpu.sync_copy(src_hbm.at[wid], src_v)
    pltpu.sync_copy(dst_hbm.at[wid], dst_v)

    neg = jnp.full((16,), -jnp.inf, jnp.float32)

    def init_body(i, _):
        m_local[pl.ds(i * 16, 16)] = neg
        return 0

    lax.fori_loop(0, N_NODES // 16, init_body, 0)

    def start_gathers(cc, b):
        pltpu.async_copy(q_hbm.at[dst_v.at[cc]], qbufs[b], semqs[b])
        pltpu.async_copy(k_hbm.at[src_v.at[cc]], kbufs[b], semks[b])

    def wait_gathers(cc, b):
        pltpu.make_async_copy(q_hbm.at[dst_v.at[cc]], qbufs[b],
                              semqs[b]).wait()
        pltpu.make_async_copy(k_hbm.at[src_v.at[cc]], kbufs[b],
                              semks[b]).wait()

    for b in range(3):
        start_gathers(b, b)

    def compute_chunk(cc, b):
        qbuf = qbufs[b]
        kbuf = kbufs[b]

        def group_body(g, _2):
            base = g * 16
            io = _iota16()
            l16 = jnp.zeros((16,), jnp.float32)
            for j in range(16):
                r = base + j
                acc = qbuf[r, pl.ds(0, 16)] * kbuf[r, pl.ds(0, 16)]
                for dv in range(1, D // 16):
                    sl = pl.ds(dv * 16, 16)
                    acc = acc + qbuf[r, sl] * kbuf[r, sl]
                for sh in (1, 2, 4, 8):
                    acc = acc + _take16(acc, io ^ sh)
                l16 = jnp.where(io == j, acc, l16)
            l16 = l16 * SCALE
            lbuf[pl.ds(cc * CH + base, 16)] = l16
            d16 = dst_v[cc, pl.ds(base, 16)]
            ks, vs = plsc.sort_key_val(d16, l16)
            vs = _seg_max_scan(ks, vs)
            isl = _last_of_run(ks)
            cur = plsc.load_gather(m_local, [ks], mask=isl)
            plsc.store_scatter(m_local, [ks], jnp.maximum(cur, vs), mask=isl)
            return 0

        lax.fori_loop(0, NG, group_body, 0)

    @pl.loop(0, NCHUNK, step=3)
    def _chunks(c):
        for b in range(3):
            cc = c + b
            live = (cc < NCHUNK) if b else True

            def do_chunk():
                wait_gathers(cc, b)
                compute_chunk(cc, b)

                @pl.when(cc + 3 < NCHUNK)
                def _():
                    start_gathers(cc + 3, b)

            if b:
                pl.when(live)(do_chunk)
            else:
                do_chunk()

    pltpu.sync_copy(lbuf, logits_hbm.at[pl.ds(wid * EPT, EPT)])
    pltpu.sync_copy(m_local, mpart_hbm.at[wid])


_sc_logits = pl.kernel(
    _sc_logits_body,
    out_type=[
        jax.ShapeDtypeStruct((N_EDGES,), jnp.float32),
        jax.ShapeDtypeStruct((NW, N_NODES), jnp.float32),
    ],
    mesh=_MESH,
    scratch_types=[
        pltpu.VMEM((NCHUNK, CH), jnp.int32),
        pltpu.VMEM((NCHUNK, CH), jnp.int32),
        pltpu.VMEM((CH, D), jnp.float32),
        pltpu.VMEM((CH, D), jnp.float32),
        pltpu.VMEM((CH, D), jnp.float32),
        pltpu.VMEM((CH, D), jnp.float32),
        pltpu.VMEM((CH, D), jnp.float32),
        pltpu.VMEM((CH, D), jnp.float32),
        pltpu.VMEM((EPT,), jnp.float32),
        pltpu.VMEM((N_NODES,), jnp.float32),
        pltpu.SemaphoreType.DMA,
        pltpu.SemaphoreType.DMA,
        pltpu.SemaphoreType.DMA,
        pltpu.SemaphoreType.DMA,
        pltpu.SemaphoreType.DMA,
        pltpu.SemaphoreType.DMA,
    ],
    compiler_params=_SC_PARAMS,
)


def _seg_add_scan(keys, vals):
    io = _iota16()
    for sh in (1, 2, 4, 8):
        idx = jnp.maximum(io - sh, 0)
        kv = _take16(keys, idx)
        vv = _take16(vals, idx)
        ok = (io >= sh) & (kv == keys)
        vals = vals + jnp.where(ok, vv, 0.0)
    return vals


def _sc_agg_body(v_hbm, srcf_hbm, dstf_hbm, dst3_hbm, logits_hbm, m_hbm,
                 zeros_hbm, agg_hbm, spart_hbm,
                 src_c0, src_c1, src_c2, dst_c0, dst_c1, dst_c2,
                 l_c0, l_c1, l_c2,
                 vbuf0, vbuf1, m_v, s_local, agg_sh,
                 semg0, semg1, semsc0, semsc1, semsm0, semsm1, semsm2):
    cid = lax.axis_index("c")
    sid = lax.axis_index("s")
    wid = cid * NS + sid
    src_cs = (src_c0, src_c1, src_c2)
    dst_cs = (dst_c0, dst_c1, dst_c2)
    l_cs = (l_c0, l_c1, l_c2)
    vbufs = (vbuf0, vbuf1)
    semgs = (semg0, semg1)
    semscs = (semsc0, semsc1)
    semsms = (semsm0, semsm1, semsm2)
    pltpu.sync_copy(m_hbm, m_v)

    zero = jnp.zeros((16,), jnp.float32)

    def init_body(i, _):
        s_local[pl.ds(i * 16, 16)] = zero
        return 0

    lax.fori_loop(0, N_NODES // 16, init_body, 0)

    pltpu.sync_copy(zeros_hbm, agg_sh.at[pl.ds(sid * ROWS_PT, ROWS_PT)])
    plsc.subcore_barrier()

    def small_descs(cc, bs):
        base_e = wid * EPT + cc * CH
        return (
            pltpu.make_async_copy(srcf_hbm.at[pl.ds(base_e, CH)],
                                  src_cs[bs], semsms[bs]),
            pltpu.make_async_copy(logits_hbm.at[pl.ds(base_e, CH)],
                                  l_cs[bs], semsms[bs]),
            pltpu.make_async_copy(dst3_hbm.at[wid * NCHUNK + cc],
                                  dst_cs[bs], semsms[bs]),
        )

    def start_smalls(cc, bs):
        for d in small_descs(cc, bs):
            d.start()

    def wait_smalls(cc, bs):
        for d in small_descs(cc, bs):
            d.wait()

    def start_gather(cc, bs, bv):
        pltpu.async_copy(v_hbm.at[src_cs[bs]], vbufs[bv], semgs[bv])

    def wait_gather(cc, bs, bv):
        pltpu.make_async_copy(v_hbm.at[src_cs[bs]], vbufs[bv],
                              semgs[bv]).wait()

    def start_scatter(bs, bv):
        pltpu.async_copy(vbufs[bv], agg_sh.at[dst_cs[bs].at[0]],
                         semscs[bv], add=True)

    def wait_scatter(bs, bv):
        pltpu.make_async_copy(vbufs[bv], agg_sh.at[dst_cs[bs].at[0]],
                              semscs[bv]).wait()

    for b in range(2):
        start_smalls(b, b)
        wait_smalls(b, b)
        start_gather(b, b, b)

    def compute_chunk(bs, bv):
        vbuf = vbufs[bv]
        l_c = l_cs[bs]
        dst_c = dst_cs[bs]

        def group_body(g, _2):
            base = g * 16
            l16 = l_c[pl.ds(base, 16)]
            d16 = dst_c[0, pl.ds(base, 16)]
            mg = plsc.load_gather(m_v, [d16])
            e16 = jnp.exp(l16 - mg)
            ks, es = plsc.sort_key_val(d16, e16)
            es = _seg_add_scan(ks, es)
            isl = _last_of_run(ks)
            cur = plsc.load_gather(s_local, [ks], mask=isl)
            plsc.store_scatter(s_local, [ks], cur + es, mask=isl)
            for j in range(16):
                r = base + j
                ebc = _take16(e16, jnp.full((16,), j, jnp.int32))
                for dv in range(D // 16):
                    sl = pl.ds(dv * 16, 16)
                    vbuf[r, sl] = vbuf[r, sl] * ebc
            return 0

        lax.fori_loop(0, NG, group_body, 0)

    @pl.loop(0, NCHUNK, step=6)
    def _chunks(c):
        for b in range(6):
            cc = c + b
            bs = b % 3
            bv = b % 2
            live = (cc < NCHUNK) if b else True

            def do_chunk():
                @pl.when(cc + 2 < NCHUNK)
                def _():
                    start_smalls(cc + 2, (bs + 2) % 3)

                wait_gather(cc, bs, bv)
                compute_chunk(bs, bv)
                start_scatter(bs, bv)

                @pl.when(cc + 2 < NCHUNK)
                def _():
                    wait_scatter(bs, bv)
                    wait_smalls(cc + 2, (bs + 2) % 3)
                    start_gather(cc + 2, (bs + 2) % 3, bv)

            if b:
                pl.when(live)(do_chunk)
            else:
                do_chunk()

    wait_scatter((NCHUNK - 2) % 3, (NCHUNK - 2) % 2)
    wait_scatter((NCHUNK - 1) % 3, (NCHUNK - 1) % 2)

    pltpu.sync_copy(s_local, spart_hbm.at[wid])
    plsc.subcore_barrier()
    sl = pl.ds(sid * ROWS_PT, ROWS_PT)
    pltpu.sync_copy(agg_sh.at[sl], agg_hbm.at[cid, sl])


_sc_agg = pl.kernel(
    _sc_agg_body,
    out_type=[
        jax.ShapeDtypeStruct((NC, N_PAD, D), jnp.float32),
        jax.ShapeDtypeStruct((NW, N_NODES), jnp.float32),
    ],
    mesh=_MESH,
    scratch_types=[
        pltpu.VMEM((CH,), jnp.int32),
        pltpu.VMEM((CH,), jnp.int32),
        pltpu.VMEM((CH,), jnp.int32),
        pltpu.VMEM((1, CH), jnp.int32),
        pltpu.VMEM((1, CH), jnp.int32),
        pltpu.VMEM((1, CH), jnp.int32),
        pltpu.VMEM((CH,), jnp.float32),
        pltpu.VMEM((CH,), jnp.float32),
        pltpu.VMEM((CH,), jnp.float32),
        pltpu.VMEM((CH, D), jnp.float32),
        pltpu.VMEM((CH, D), jnp.float32),
        pltpu.VMEM((N_NODES,), jnp.float32),
        pltpu.VMEM((N_NODES,), jnp.float32),
        pltpu.VMEM_SHARED((N_PAD, D), jnp.float32),
        pltpu.SemaphoreType.DMA,
        pltpu.SemaphoreType.DMA,
        pltpu.SemaphoreType.DMA,
        pltpu.SemaphoreType.DMA,
        pltpu.SemaphoreType.DMA,
        pltpu.SemaphoreType.DMA,
        pltpu.SemaphoreType.DMA,
    ],
    compiler_params=_SC_PARAMS,
)


def _proj_kernel(x_ref, wq_ref, bq_ref, wk_ref, bk_ref, wv_ref, bv_ref,
                 ws_ref, bs_ref, q_ref, k_ref, vx_ref, s_ref):
    x = x_ref[...]
    q_ref[...] = jnp.dot(x, wq_ref[...],
                         preferred_element_type=jnp.float32) + bq_ref[...]
    k_ref[...] = jnp.dot(x, wk_ref[...],
                         preferred_element_type=jnp.float32) + bk_ref[...]
    vx_ref[...] = jnp.dot(x, wv_ref[...],
                          preferred_element_type=jnp.float32) + bv_ref[...]
    s_ref[...] = jnp.dot(x, ws_ref[...],
                         preferred_element_type=jnp.float32) + bs_ref[...]


def _proj(x, wq, bq, wk, bk, wv, bv, ws, bs):
    n = x.shape[0]
    nb = 2000
    wspec = pl.BlockSpec((D, D), lambda i: (0, 0))
    bspec = pl.BlockSpec((1, D), lambda i: (0, 0))
    return pl.pallas_call(
        _proj_kernel,
        grid=(n // nb,),
        in_specs=[pl.BlockSpec((nb, D), lambda i: (i, 0)),
                  wspec, bspec, wspec, bspec, wspec, bspec, wspec, bspec],
        out_specs=[pl.BlockSpec((nb, D), lambda i: (i, 0)),
                   pl.BlockSpec((nb, D), lambda i: (i, 0)),
                   pl.BlockSpec((nb, D), lambda i: (i, 0)),
                   pl.BlockSpec((nb, D), lambda i: (i, 0))],
        out_shape=[jax.ShapeDtypeStruct((n, D), jnp.float32),
                   jax.ShapeDtypeStruct((n, D), jnp.float32),
                   jax.ShapeDtypeStruct((n, D), jnp.float32),
                   jax.ShapeDtypeStruct((n, D), jnp.float32)],
    )(x, wq, bq.reshape(1, D), wk, bk.reshape(1, D), wv, bv.reshape(1, D),
      ws, bs.reshape(1, D))


def _maxreduce_kernel(mp_ref, m_ref):
    m = jnp.max(mp_ref[...], axis=0, keepdims=True)
    m_ref[...] = jnp.where(jnp.isfinite(m), m, 0.0)


def _maxreduce(mpart):
    return pl.pallas_call(
        _maxreduce_kernel,
        out_shape=jax.ShapeDtypeStruct((1, N_NODES), jnp.float32),
    )(mpart)


def _sumreduce_kernel(sp_ref, s_ref):
    s_ref[...] = jnp.sum(sp_ref[...], axis=0)[:, None]


def _sumreduce(spart):
    return pl.pallas_call(
        _sumreduce_kernel,
        out_shape=jax.ShapeDtypeStruct((N_NODES, 1), jnp.float32),
    )(spart)


def _epilogue_h(agg, scol, skip):
    a = agg[0] + agg[1]
    return jax.nn.relu(a / (scol + 1e-16) + skip)


def _epi_proj_kernel(agg_ref, sp_ref, skip_ref, wq_ref, bq_ref, wk_ref,
                     bk_ref, wv_ref, bv_ref, ws_ref, bs_ref,
                     q_ref, k_ref, vx_ref, s_ref):
    h = _epilogue_h(agg_ref[...], sp_ref[...], skip_ref[...])
    q_ref[...] = jnp.dot(h, wq_ref[...],
                         preferred_element_type=jnp.float32) + bq_ref[...]
    k_ref[...] = jnp.dot(h, wk_ref[...],
                         preferred_element_type=jnp.float32) + bk_ref[...]
    vx_ref[...] = jnp.dot(h, wv_ref[...],
                          preferred_element_type=jnp.float32) + bv_ref[...]
    s_ref[...] = jnp.dot(h, ws_ref[...],
                         preferred_element_type=jnp.float32) + bs_ref[...]


def _epi_proj(agg, spart, skip, wq, bq, wk, bk, wv, bv, ws, bs):
    n = skip.shape[0]
    nb = 2000
    wspec = pl.BlockSpec((D, D), lambda i: (0, 0))
    bspec = pl.BlockSpec((1, D), lambda i: (0, 0))
    return pl.pallas_call(
        _epi_proj_kernel,
        grid=(n // nb,),
        in_specs=[pl.BlockSpec((NC, nb, D), lambda i: (0, i, 0)),
                  pl.BlockSpec((nb, 1), lambda i: (i, 0)),
                  pl.BlockSpec((nb, D), lambda i: (i, 0)),
                  wspec, bspec, wspec, bspec, wspec, bspec, wspec, bspec],
        out_specs=[pl.BlockSpec((nb, D), lambda i: (i, 0)),
                   pl.BlockSpec((nb, D), lambda i: (i, 0)),
                   pl.BlockSpec((nb, D), lambda i: (i, 0)),
                   pl.BlockSpec((nb, D), lambda i: (i, 0))],
        out_shape=[jax.ShapeDtypeStruct((n, D), jnp.float32),
                   jax.ShapeDtypeStruct((n, D), jnp.float32),
                   jax.ShapeDtypeStruct((n, D), jnp.float32),
                   jax.ShapeDtypeStruct((n, D), jnp.float32)],
    )(agg, spart, skip, wq, bq.reshape(1, D), wk, bk.reshape(1, D),
      wv, bv.reshape(1, D), ws, bs.reshape(1, D))


def _pool_mlp_kernel(agg_ref, sp_ref, skip_ref, batch_ref, w1_ref, b1_ref,
                     w2_ref, b2_ref, o_ref, acc_ref):
    i = pl.program_id(0)

    @pl.when(i == 0)
    def _():
        acc_ref[...] = jnp.zeros_like(acc_ref)

    h2 = _epilogue_h(agg_ref[...], sp_ref[...], skip_ref[...])
    b = batch_ref[0, 0]
    onehot = (jax.lax.broadcasted_iota(jnp.int32, (G, b.shape[0]), 0)
              == b[None, :]).astype(jnp.float32)
    acc_ref[...] += jnp.dot(onehot, h2, preferred_element_type=jnp.float32)

    @pl.when(i == pl.num_programs(0) - 1)
    def _():
        gact = jax.nn.relu(
            jnp.dot(acc_ref[...], w1_ref[...],
                    preferred_element_type=jnp.float32) + b1_ref[...])
        o_ref[...] = jnp.dot(gact, w2_ref[...],
                             preferred_element_type=jnp.float32) + b2_ref[...]


def _pool_mlp(agg, spart, skip, batch, w1, b1, w2, b2):
    n = skip.shape[0]
    nb = 2000
    return pl.pallas_call(
        _pool_mlp_kernel,
        grid=(n // nb,),
        in_specs=[
            pl.BlockSpec((NC, nb, D), lambda i: (0, i, 0)),
            pl.BlockSpec((nb, 1), lambda i: (i, 0)),
            pl.BlockSpec((nb, D), lambda i: (i, 0)),
            pl.BlockSpec((1, 1, nb), lambda i: (i, 0, 0)),
            pl.BlockSpec(w1.shape, lambda i: (0, 0)),
            pl.BlockSpec((1, D), lambda i: (0, 0)),
            pl.BlockSpec(w2.shape, lambda i: (0, 0)),
            pl.BlockSpec((1, D), lambda i: (0, 0)),
        ],
        out_specs=pl.BlockSpec((G, D), lambda i: (0, 0)),
        out_shape=jax.ShapeDtypeStruct((G, D), jnp.float32),
        scratch_shapes=[pltpu.VMEM((G, D), jnp.float32)],
    )(agg, spart, skip, batch.reshape(n // nb, 1, nb), w1, b1.reshape(1, D),
      w2, b2.reshape(1, D))


def _edge_layer_sc(q, k, v, src_f, dst_f, src_t, dst_t, dst3, zeros_rows):
    logits, mpart = _sc_logits(q, k, src_t, dst_t)
    m = _maxreduce(mpart).reshape(N_NODES)
    agg, spart = _sc_agg(v, src_f, dst_f, dst3, logits, m, zeros_rows)
    return agg, _sumreduce(spart)


def kernel(x, edge_index, batch,
           Wq0, bq0, Wk0, bk0, Wv0, bv0, Ws0, bs0,
           Wq1, bq1, Wk1, bk1, Wv1, bv1, Ws1, bs1,
           W1, b1, W2, b2):
    src_f = edge_index[0]
    dst_f = edge_index[1]
    src_t = edge_index[0].reshape(NW, NCHUNK, CH)
    dst_t = edge_index[1].reshape(NW, NCHUNK, CH)
    dst3 = edge_index[1].reshape(NW * NCHUNK, 1, CH)
    zeros_rows = jnp.zeros((ROWS_PT, D), jnp.float32)

    q0, k0, vx0, skip0 = _proj(x, Wq0, bq0, Wk0, bk0, Wv0, bv0, Ws0, bs0)
    agg0, sp0 = _edge_layer_sc(q0, k0, vx0, src_f, dst_f, src_t, dst_t,
                               dst3, zeros_rows)

    q1, k1, vx1, skip1 = _epi_proj(agg0, sp0, skip0, Wq1, bq1, Wk1, bk1,
                                   Wv1, bv1, Ws1, bs1)
    agg1, sp1 = _edge_layer_sc(q1, k1, vx1, src_f, dst_f, src_t, dst_t,
                               dst3, zeros_rows)

    return _pool_mlp(agg1, sp1, skip1, batch, W1, b1, W2, b2)
